# baseline (speedup 1.0000x reference)
import functools

import jax
import jax.numpy as jnp
from jax import lax
from jax.experimental import pallas as pl
from jax.experimental.pallas import tpu as pltpu
from jax.experimental.pallas import tpu_sc as plsc

N = 10000
E = 320000
IN_SIZE = 128
HID = 16
HEADS = 8
OUT_SIZE = 16

NP = 10112
K = 128
SUP = 4
NCHUNKP = 2560
CPW = NCHUNKP // 32
CPW1 = NCHUNKP // 16
D1 = 80
D2 = 32
TAB1 = 40016
TAB2 = 10016


def _edge_pass_l1(idx2, packed, adst_tab, zinit):
  mesh = plsc.VectorSubcoreMesh(core_axis_name="c", subcore_axis_name="s")

  @functools.partial(
      pl.kernel,
      out_type=jax.ShapeDtypeStruct((2, NP, D1), jnp.float32),
      mesh=mesh,
      scratch_types=[
          pltpu.VMEM((SUP, 2 * K), jnp.int32),
          pltpu.VMEM((K,), jnp.int32),
          pltpu.VMEM((K,), jnp.int32),
          pltpu.VMEM((K, D1), jnp.float32),
          pltpu.VMEM((K, D1), jnp.float32),
          pltpu.VMEM((TAB1,), jnp.float32),
          pltpu.VMEM_SHARED((NP, D1), jnp.float32),
          pltpu.SemaphoreType.DMA,
      ],
      compiler_params=pltpu.CompilerParams(
          needs_layout_passes=False, use_tc_tiling_on_sc=False),
  )
  def kern(idx2_hbm, packed_hbm, adst_hbm, zero_hbm, out_hbm,
           comb_v, src2_v, dst_v, rows_v, stage_v, tab_v, acc_sh, sem):
    cid = lax.axis_index("c")
    sid = lax.axis_index("s")
    rpw = NP // 16
    pltpu.sync_copy(zero_hbm.at[pl.ds(sid * rpw, rpw)],
                    acc_sh.at[pl.ds(sid * rpw, rpw)])
    pltpu.sync_copy(adst_hbm.at[pl.ds(cid * TAB1, TAB1)], tab_v)
    plsc.subcore_barrier()
    iota = lax.iota(jnp.int32, 16)
    lane_lt4 = iota < 4
    row_off = cid * N
    c0 = sid * CPW1

    def chunk_body(i, carry):
      pltpu.sync_copy(idx2_hbm.at[pl.ds(c0 + SUP * i, SUP)], comb_v)
      for cc in range(SUP):
        for t in range(K // 16):
          src2_v[pl.ds(t * 16, 16)] = (
              comb_v[cc, pl.ds(t * 16, 16)] + row_off)
          dst_v[pl.ds(t * 16, 16)] = comb_v[cc, pl.ds(K + t * 16, 16)]
        pltpu.async_copy(packed_hbm.at[src2_v], rows_v, sem).wait()

        def group(g, c2):
          dstg = dst_v[pl.ds(g * 16, 16)]
          d4 = dstg * 4
          for k in range(16):
            e = g * 16 + k
            db = jnp.take_along_axis(
                d4, jnp.full((16,), k, jnp.int32), axis=0,
                mode="promise_in_bounds")
            adstv = plsc.load_gather(tab_v, [db + iota])
            asrcv = rows_v[e, pl.ds(0, 16)]
            z = asrcv + adstv
            z = jnp.maximum(z, z * 0.2)
            w = jnp.where(lane_lt4, jnp.exp(z), 0.0)
            stage_v[e, pl.ds(0, 16)] = w
            for j in range(4):
              bj = jnp.take_along_axis(
                  w, jnp.full((16,), j, jnp.int32), axis=0,
                  mode="promise_in_bounds")
              stage_v[e, pl.ds(16 + j * 16, 16)] = (
                  rows_v[e, pl.ds(16 + j * 16, 16)] * bj)
          return c2

        lax.fori_loop(0, K // 16, group, 0)
        pltpu.sync_copy(stage_v, acc_sh.at[dst_v], add=True)

      return carry

    lax.fori_loop(0, CPW1 // SUP, chunk_body, 0)
    plsc.subcore_barrier()
    pltpu.sync_copy(acc_sh.at[pl.ds(sid * rpw, rpw)],
                    out_hbm.at[cid, pl.ds(sid * rpw, rpw)])

  return kern(idx2, packed, adst_tab, zinit)


def _edge_pass_l2(idx2, packed, adst_tab, zinit):
  mesh = plsc.VectorSubcoreMesh(core_axis_name="c", subcore_axis_name="s")

  @functools.partial(
      pl.kernel,
      out_type=jax.ShapeDtypeStruct((2, NP, D2), jnp.float32),
      mesh=mesh,
      scratch_types=[
          pltpu.VMEM((SUP, 2 * K), jnp.int32),
          pltpu.VMEM((K,), jnp.int32),
          pltpu.VMEM((K,), jnp.int32),
          pltpu.VMEM((K, D2), jnp.float32),
          pltpu.VMEM((K, D2), jnp.float32),
          pltpu.VMEM((TAB2,), jnp.float32),
          pltpu.VMEM_SHARED((NP, D2), jnp.float32),
          pltpu.SemaphoreType.DMA,
      ],
      compiler_params=pltpu.CompilerParams(
          needs_layout_passes=False, use_tc_tiling_on_sc=False),
  )
  def kern(idx2_hbm, packed_hbm, adst_hbm, zero_hbm, out_hbm,
           comb_v, src_v, dst_v, rows_v, stage_v, tab_v, acc_sh, sem):
    cid = lax.axis_index("c")
    sid = lax.axis_index("s")
    wid = sid * 2 + cid
    rpw = NP // 16
    pltpu.sync_copy(zero_hbm.at[pl.ds(sid * rpw, rpw)],
                    acc_sh.at[pl.ds(sid * rpw, rpw)])
    pltpu.sync_copy(adst_hbm, tab_v)
    plsc.subcore_barrier()
    iota = lax.iota(jnp.int32, 16)

    c0 = wid * CPW

    def chunk_body(i, carry):
      pltpu.sync_copy(idx2_hbm.at[pl.ds(c0 + SUP * i, SUP)], comb_v)
      for cc in range(SUP):
        for t in range(K // 16):
          src_v[pl.ds(t * 16, 16)] = comb_v[cc, pl.ds(t * 16, 16)]
          dst_v[pl.ds(t * 16, 16)] = comb_v[cc, pl.ds(K + t * 16, 16)]
        pltpu.async_copy(packed_hbm.at[src_v], rows_v, sem).wait()

        def group(g, c2):
          e16 = g * 16 + iota
          dstg = dst_v[pl.ds(g * 16, 16)]
          adstv = plsc.load_gather(tab_v, [dstg])
          asrcv = plsc.load_gather(
              rows_v, [e16, jnp.full((16,), 16, jnp.int32)])
          z = asrcv + adstv
          z = jnp.maximum(z, z * 0.2)
          w = jnp.exp(z)
          plsc.store_scatter(
              stage_v, [e16, jnp.full((16,), 16, jnp.int32)], w)
          for k in range(16):
            e = g * 16 + k
            bk = jnp.take_along_axis(
                w, jnp.full((16,), k, jnp.int32), axis=0,
                mode="promise_in_bounds")
            stage_v[e, pl.ds(0, 16)] = rows_v[e, pl.ds(0, 16)] * bk
          return c2

        lax.fori_loop(0, K // 16, group, 0)
        pltpu.sync_copy(stage_v, acc_sh.at[dst_v], add=True)
      return carry

    lax.fori_loop(0, CPW // SUP, chunk_body, 0)
    plsc.subcore_barrier()
    pltpu.sync_copy(acc_sh.at[pl.ds(sid * rpw, rpw)],
                    out_hbm.at[cid, pl.ds(sid * rpw, rpw)])

  return kern(idx2, packed, adst_tab, zinit)


def _prep1(x, W1, Asrc, Adst, R8):
  def body(x_ref, w1_ref, as_ref, ad_ref, r8_ref,
           packed_ref, adst_ref, self_ref):
    h = jnp.dot(x_ref[...], w1_ref[...], preferred_element_type=jnp.float32)
    asrc = jnp.dot(h, as_ref[...], preferred_element_type=jnp.float32)
    adst = jnp.dot(h, ad_ref[...], preferred_element_type=jnp.float32)
    z = asrc + adst
    w = jnp.exp(jnp.maximum(z, 0.2 * z))
    wrep = jnp.dot(w, r8_ref[...], preferred_element_type=jnp.float32)
    zpad = jnp.zeros((h.shape[0], 12), jnp.float32)
    for q in range(2):
      packed_ref[q, :, 0:4] = asrc[:, 4 * q:4 * q + 4]
      packed_ref[q, :, 4:16] = zpad
      packed_ref[q, :, 16:80] = h[:, 64 * q:64 * q + 64]
    self_ref[:, 0:128] = h * wrep
    self_ref[:, 128:136] = asrc * 0.0 + w
    self_ref[:, 136:144] = jnp.zeros_like(w)
    adst_ref[...] = adst

  return pl.pallas_call(
      body,
      out_shape=[
          jax.ShapeDtypeStruct((2, N, D1), jnp.float32),
          jax.ShapeDtypeStruct((N, HEADS), jnp.float32),
          jax.ShapeDtypeStruct((N, 144), jnp.float32),
      ],
  )(x, W1, Asrc, Adst, R8)


def _prep2(p0, p1, si1, b1, W2, att_s2, att_d2, R8):
  def body(p0_ref, p1_ref, si_ref, b1_ref, w2_ref, as_ref, ad_ref, r8_ref,
           packed_ref, adst_ref, self_ref):
    p0 = p0_ref[...]
    p1 = p1_ref[...]
    si = si_ref[...]
    m = jnp.concatenate([p0[:, 16:80], p1[:, 16:80]], axis=1) + si[:, 0:128]
    s = jnp.concatenate([p0[:, 0:4], p1[:, 0:4]], axis=1) + si[:, 128:136]
    srep = jnp.dot(s, r8_ref[...], preferred_element_type=jnp.float32)
    o1 = m / (srep + 1e-16) + b1_ref[...]
    e1 = jnp.where(o1 > 0, o1, jnp.exp(o1) - 1.0)
    h2 = jnp.dot(e1, w2_ref[...], preferred_element_type=jnp.float32)
    as2 = jnp.sum(h2 * as_ref[...], axis=1, keepdims=True)
    ad2 = jnp.sum(h2 * ad_ref[...], axis=1, keepdims=True)
    z = as2 + ad2
    w = jnp.exp(jnp.maximum(z, 0.2 * z))
    zpad = jnp.zeros((h2.shape[0], 15), jnp.float32)
    packed_ref[:, 0:16] = h2
    packed_ref[:, 16:17] = as2
    packed_ref[:, 17:32] = zpad
    self_ref[:, 0:16] = h2 * w
    self_ref[:, 16:17] = w
    self_ref[:, 17:32] = zpad
    adst_ref[...] = ad2

  return pl.pallas_call(
      body,
      out_shape=[
          jax.ShapeDtypeStruct((N, D2), jnp.float32),
          jax.ShapeDtypeStruct((N, 1), jnp.float32),
          jax.ShapeDtypeStruct((N, D2), jnp.float32),
      ],
  )(p0, p1, si1, b1, W2, att_s2, att_d2, R8)


def _final(q0, q1, si2, b2):
  def body(q0_ref, q1_ref, si_ref, b2_ref, out_ref):
    acc = q0_ref[...] + q1_ref[...] + si_ref[...]
    o = acc[:, 0:16] / (acc[:, 16:17] + 1e-16) + b2_ref[...]
    mx = jnp.max(o, axis=1, keepdims=True)
    lse = jnp.log(jnp.sum(jnp.exp(o - mx), axis=1, keepdims=True))
    out_ref[...] = o - mx - lse

  return pl.pallas_call(
      body,
      out_shape=jax.ShapeDtypeStruct((N, OUT_SIZE), jnp.float32),
  )(q0, q1, si2, b2)


def kernel(x, edge_index, W1, att_src1, att_dst1, b1, W2, att_src2, att_dst2,
           b2):
  npadE = NCHUNKP * K - E
  srcp = jnp.pad(edge_index[0].astype(jnp.int32), (0, npadE))
  dstp = jnp.pad(edge_index[1].astype(jnp.int32), (0, npadE),
                 constant_values=N)
  idx2 = jnp.concatenate([
      srcp.reshape(NCHUNKP, 1, K),
      dstp.reshape(NCHUNKP, 1, K),
  ], axis=1).reshape(NCHUNKP, 2 * K)

  eye8 = jnp.eye(HEADS, dtype=jnp.float32)
  Asrc = (att_src1[:, :, None] * eye8[:, None, :]).reshape(IN_SIZE, HEADS)
  Adst = (att_dst1[:, :, None] * eye8[:, None, :]).reshape(IN_SIZE, HEADS)
  R8 = jnp.kron(eye8, jnp.ones((1, HID), jnp.float32))

  packed1, adst1, si1 = _prep1(x, W1, Asrc, Adst, R8)
  tab1 = jnp.concatenate([
      jnp.pad(adst1[:, 0:4].reshape(-1), (0, TAB1 - N * 4)),
      jnp.pad(adst1[:, 4:8].reshape(-1), (0, TAB1 - N * 4)),
  ])
  z1 = jnp.zeros((NP, D1), jnp.float32)
  acc1 = _edge_pass_l1(idx2, packed1.reshape(2 * N, D1), tab1, z1)

  packed2, adst2, si2 = _prep2(
      acc1[0, :N], acc1[1, :N], si1, b1.reshape(1, IN_SIZE), W2,
      att_src2, att_dst2, R8)
  tab2 = jnp.pad(adst2.reshape(-1), (0, TAB2 - N))
  z2 = jnp.zeros((NP, D2), jnp.float32)
  acc2 = _edge_pass_l2(idx2, packed2, tab2, z2)

  return _final(acc2[0, :N], acc2[1, :N], si2, b2.reshape(1, OUT_SIZE))


# speedup vs baseline: 1.2395x; 1.2395x over previous
import functools

import jax
import jax.numpy as jnp
from jax import lax
from jax.experimental import pallas as pl
from jax.experimental.pallas import tpu as pltpu
from jax.experimental.pallas import tpu_sc as plsc

N = 10000
E = 320000
IN_SIZE = 128
HID = 16
HEADS = 8
OUT_SIZE = 16

NP = 10112
K = 128
NCHUNK = E // K
NW = 32
CPW = -(-NCHUNK // NW)
CPW1 = -(-NCHUNK // 16)
D1 = 80
D2 = 32
TAB1 = 40016
TAB2 = 10016


def _edge_pass_l1(idx2, packed, adst_tab, zinit):
  mesh = plsc.VectorSubcoreMesh(core_axis_name="c", subcore_axis_name="s")

  @functools.partial(
      pl.kernel,
      out_type=jax.ShapeDtypeStruct((2, NP, D1), jnp.float32),
      mesh=mesh,
      scratch_types=[
          pltpu.VMEM((2 * K,), jnp.int32),
          pltpu.VMEM((K,), jnp.int32),
          pltpu.VMEM((K,), jnp.int32),
          pltpu.VMEM((K, D1), jnp.float32),
          pltpu.VMEM((K, D1), jnp.float32),
          pltpu.VMEM((TAB1,), jnp.float32),
          pltpu.VMEM_SHARED((NP, D1), jnp.float32),
          pltpu.SemaphoreType.DMA,
      ],
      compiler_params=pltpu.CompilerParams(
          needs_layout_passes=False, use_tc_tiling_on_sc=False),
  )
  def kern(idx2_hbm, packed_hbm, adst_hbm, zero_hbm, out_hbm,
           comb_v, src2_v, dst_v, rows_v, stage_v, tab_v, acc_sh, sem):
    cid = lax.axis_index("c")
    sid = lax.axis_index("s")
    rpw = NP // 16
    pltpu.sync_copy(zero_hbm.at[pl.ds(sid * rpw, rpw)],
                    acc_sh.at[pl.ds(sid * rpw, rpw)])
    pltpu.sync_copy(adst_hbm.at[pl.ds(cid * TAB1, TAB1)], tab_v)
    plsc.subcore_barrier()
    iota = lax.iota(jnp.int32, 16)
    lane_lt4 = iota < 4
    row_off = cid * N

    def chunk_body(i, carry):
      chunk = sid + 16 * i

      @pl.when(chunk < NCHUNK)
      def _():
        pltpu.sync_copy(idx2_hbm.at[chunk], comb_v)
        for t in range(K // 16):
          src2_v[pl.ds(t * 16, 16)] = comb_v[pl.ds(t * 16, 16)] + row_off
          dst_v[pl.ds(t * 16, 16)] = comb_v[pl.ds(K + t * 16, 16)]
        pltpu.async_copy(packed_hbm.at[src2_v], rows_v, sem).wait()

        def group(g, c2):
          dstg = dst_v[pl.ds(g * 16, 16)]
          d4 = dstg * 4
          for k in range(16):
            e = g * 16 + k
            db = jnp.take_along_axis(
                d4, jnp.full((16,), k, jnp.int32), axis=0,
                mode="promise_in_bounds")
            adstv = plsc.load_gather(tab_v, [db + iota])
            asrcv = rows_v[e, pl.ds(0, 16)]
            z = asrcv + adstv
            z = jnp.maximum(z, z * 0.2)
            w = jnp.where(lane_lt4, jnp.exp(z), 0.0)
            stage_v[e, pl.ds(0, 16)] = w
            for j in range(4):
              bj = jnp.take_along_axis(
                  w, jnp.full((16,), j, jnp.int32), axis=0,
                  mode="promise_in_bounds")
              stage_v[e, pl.ds(16 + j * 16, 16)] = (
                  rows_v[e, pl.ds(16 + j * 16, 16)] * bj)
          return c2

        lax.fori_loop(0, K // 16, group, 0)
        pltpu.sync_copy(stage_v, acc_sh.at[dst_v], add=True)

      return carry

    lax.fori_loop(0, CPW1, chunk_body, 0)
    plsc.subcore_barrier()
    pltpu.sync_copy(acc_sh.at[pl.ds(sid * rpw, rpw)],
                    out_hbm.at[cid, pl.ds(sid * rpw, rpw)])

  return kern(idx2, packed, adst_tab, zinit)


def _edge_pass_l2(idx2, packed, adst_tab, zinit):
  mesh = plsc.VectorSubcoreMesh(core_axis_name="c", subcore_axis_name="s")

  @functools.partial(
      pl.kernel,
      out_type=jax.ShapeDtypeStruct((2, NP, D2), jnp.float32),
      mesh=mesh,
      scratch_types=[
          pltpu.VMEM((2 * K,), jnp.int32),
          pltpu.VMEM((K,), jnp.int32),
          pltpu.VMEM((K,), jnp.int32),
          pltpu.VMEM((K, D2), jnp.float32),
          pltpu.VMEM((K, D2), jnp.float32),
          pltpu.VMEM((TAB2,), jnp.float32),
          pltpu.VMEM_SHARED((NP, D2), jnp.float32),
          pltpu.SemaphoreType.DMA,
      ],
      compiler_params=pltpu.CompilerParams(
          needs_layout_passes=False, use_tc_tiling_on_sc=False),
  )
  def kern(idx2_hbm, packed_hbm, adst_hbm, zero_hbm, out_hbm,
           comb_v, src_v, dst_v, rows_v, stage_v, tab_v, acc_sh, sem):
    cid = lax.axis_index("c")
    sid = lax.axis_index("s")
    wid = sid * 2 + cid
    rpw = NP // 16
    pltpu.sync_copy(zero_hbm.at[pl.ds(sid * rpw, rpw)],
                    acc_sh.at[pl.ds(sid * rpw, rpw)])
    pltpu.sync_copy(adst_hbm, tab_v)
    plsc.subcore_barrier()
    iota = lax.iota(jnp.int32, 16)

    def chunk_body(i, carry):
      chunk = wid + NW * i

      @pl.when(chunk < NCHUNK)
      def _():
        pltpu.sync_copy(idx2_hbm.at[chunk], comb_v)
        for t in range(K // 16):
          src_v[pl.ds(t * 16, 16)] = comb_v[pl.ds(t * 16, 16)]
          dst_v[pl.ds(t * 16, 16)] = comb_v[pl.ds(K + t * 16, 16)]
        pltpu.async_copy(packed_hbm.at[src_v], rows_v, sem).wait()

        def group(g, c2):
          e16 = g * 16 + iota
          dstg = dst_v[pl.ds(g * 16, 16)]
          adstv = plsc.load_gather(tab_v, [dstg])
          asrcv = plsc.load_gather(
              rows_v, [e16, jnp.full((16,), 16, jnp.int32)])
          z = asrcv + adstv
          z = jnp.maximum(z, z * 0.2)
          w = jnp.exp(z)
          plsc.store_scatter(
              stage_v, [e16, jnp.full((16,), 16, jnp.int32)], w)
          for k in range(16):
            e = g * 16 + k
            bk = jnp.take_along_axis(
                w, jnp.full((16,), k, jnp.int32), axis=0,
                mode="promise_in_bounds")
            stage_v[e, pl.ds(0, 16)] = rows_v[e, pl.ds(0, 16)] * bk
          return c2

        lax.fori_loop(0, K // 16, group, 0)
        pltpu.sync_copy(stage_v, acc_sh.at[dst_v], add=True)

      return carry

    lax.fori_loop(0, CPW, chunk_body, 0)
    plsc.subcore_barrier()
    pltpu.sync_copy(acc_sh.at[pl.ds(sid * rpw, rpw)],
                    out_hbm.at[cid, pl.ds(sid * rpw, rpw)])

  return kern(idx2, packed, adst_tab, zinit)


def _prep1(x, W1, Asrc, Adst, R8):
  def body(x_ref, w1_ref, as_ref, ad_ref, r8_ref,
           packed_ref, adst_ref, self_ref):
    h = jnp.dot(x_ref[...], w1_ref[...], preferred_element_type=jnp.float32)
    asrc = jnp.dot(h, as_ref[...], preferred_element_type=jnp.float32)
    adst = jnp.dot(h, ad_ref[...], preferred_element_type=jnp.float32)
    z = asrc + adst
    w = jnp.exp(jnp.maximum(z, 0.2 * z))
    wrep = jnp.dot(w, r8_ref[...], preferred_element_type=jnp.float32)
    zpad = jnp.zeros((h.shape[0], 12), jnp.float32)
    for q in range(2):
      packed_ref[q, :, 0:4] = asrc[:, 4 * q:4 * q + 4]
      packed_ref[q, :, 4:16] = zpad
      packed_ref[q, :, 16:80] = h[:, 64 * q:64 * q + 64]
    self_ref[:, 0:128] = h * wrep
    self_ref[:, 128:136] = asrc * 0.0 + w
    self_ref[:, 136:144] = jnp.zeros_like(w)
    adst_ref[...] = adst

  return pl.pallas_call(
      body,
      out_shape=[
          jax.ShapeDtypeStruct((2, N, D1), jnp.float32),
          jax.ShapeDtypeStruct((N, HEADS), jnp.float32),
          jax.ShapeDtypeStruct((N, 144), jnp.float32),
      ],
  )(x, W1, Asrc, Adst, R8)


def _prep2(p0, p1, si1, b1, W2, att_s2, att_d2, R8):
  def body(p0_ref, p1_ref, si_ref, b1_ref, w2_ref, as_ref, ad_ref, r8_ref,
           packed_ref, adst_ref, self_ref):
    p0 = p0_ref[...]
    p1 = p1_ref[...]
    si = si_ref[...]
    m = jnp.concatenate([p0[:, 16:80], p1[:, 16:80]], axis=1) + si[:, 0:128]
    s = jnp.concatenate([p0[:, 0:4], p1[:, 0:4]], axis=1) + si[:, 128:136]
    srep = jnp.dot(s, r8_ref[...], preferred_element_type=jnp.float32)
    o1 = m / (srep + 1e-16) + b1_ref[...]
    e1 = jnp.where(o1 > 0, o1, jnp.exp(o1) - 1.0)
    h2 = jnp.dot(e1, w2_ref[...], preferred_element_type=jnp.float32)
    as2 = jnp.sum(h2 * as_ref[...], axis=1, keepdims=True)
    ad2 = jnp.sum(h2 * ad_ref[...], axis=1, keepdims=True)
    z = as2 + ad2
    w = jnp.exp(jnp.maximum(z, 0.2 * z))
    zpad = jnp.zeros((h2.shape[0], 15), jnp.float32)
    packed_ref[:, 0:16] = h2
    packed_ref[:, 16:17] = as2
    packed_ref[:, 17:32] = zpad
    self_ref[:, 0:16] = h2 * w
    self_ref[:, 16:17] = w
    self_ref[:, 17:32] = zpad
    adst_ref[...] = ad2

  return pl.pallas_call(
      body,
      out_shape=[
          jax.ShapeDtypeStruct((N, D2), jnp.float32),
          jax.ShapeDtypeStruct((N, 1), jnp.float32),
          jax.ShapeDtypeStruct((N, D2), jnp.float32),
      ],
  )(p0, p1, si1, b1, W2, att_s2, att_d2, R8)


def _final(q0, q1, si2, b2):
  def body(q0_ref, q1_ref, si_ref, b2_ref, out_ref):
    acc = q0_ref[...] + q1_ref[...] + si_ref[...]
    o = acc[:, 0:16] / (acc[:, 16:17] + 1e-16) + b2_ref[...]
    mx = jnp.max(o, axis=1, keepdims=True)
    lse = jnp.log(jnp.sum(jnp.exp(o - mx), axis=1, keepdims=True))
    out_ref[...] = o - mx - lse

  return pl.pallas_call(
      body,
      out_shape=jax.ShapeDtypeStruct((N, OUT_SIZE), jnp.float32),
  )(q0, q1, si2, b2)


def kernel(x, edge_index, W1, att_src1, att_dst1, b1, W2, att_src2, att_dst2,
           b2):
  idx2 = jnp.concatenate([
      edge_index[0].astype(jnp.int32).reshape(NCHUNK, 1, K),
      edge_index[1].astype(jnp.int32).reshape(NCHUNK, 1, K),
  ], axis=1).reshape(NCHUNK, 2 * K)

  eye8 = jnp.eye(HEADS, dtype=jnp.float32)
  Asrc = (att_src1[:, :, None] * eye8[:, None, :]).reshape(IN_SIZE, HEADS)
  Adst = (att_dst1[:, :, None] * eye8[:, None, :]).reshape(IN_SIZE, HEADS)
  R8 = jnp.kron(eye8, jnp.ones((1, HID), jnp.float32))

  packed1, adst1, si1 = _prep1(x, W1, Asrc, Adst, R8)
  tab1 = jnp.concatenate([
      jnp.pad(adst1[:, 0:4].reshape(-1), (0, TAB1 - N * 4)),
      jnp.pad(adst1[:, 4:8].reshape(-1), (0, TAB1 - N * 4)),
  ])
  z1 = jnp.zeros((NP, D1), jnp.float32)
  acc1 = _edge_pass_l1(idx2, packed1.reshape(2 * N, D1), tab1, z1)

  packed2, adst2, si2 = _prep2(
      acc1[0, :N], acc1[1, :N], si1, b1.reshape(1, IN_SIZE), W2,
      att_src2, att_dst2, R8)
  tab2 = jnp.pad(adst2.reshape(-1), (0, TAB2 - N))
  z2 = jnp.zeros((NP, D2), jnp.float32)
  acc2 = _edge_pass_l2(idx2, packed2, tab2, z2)

  return _final(acc2[0, :N], acc2[1, :N], si2, b2.reshape(1, OUT_SIZE))
